# baseline (speedup 1.0000x reference)
import functools

import jax
import jax.numpy as jnp
from jax import lax
from jax.experimental import pallas as pl
from jax.experimental.pallas import tpu as pltpu
from jax.experimental.pallas import tpu_sc as plsc

BATCH = 16384
EMBED = 32
NUSERS = 1000000
NC = 2
NS = 16
L = 16
NW = NC * NS
BPW = BATCH // NW
IDX_CHUNK = 128
NP = BPW // IDX_CHUNK
NCH = BPW // L


def _gmf_body(users_hbm, movies_hbm, ut_hbm, mt_hbm, wb_hbm, bb_hbm, out_hbm,
              uidx_v, midx_v, eidx_v, ucols_v, mrows_v, w_v, b_v, o_v,
              sem, esem):
    wid = lax.axis_index("s") * NC + lax.axis_index("c")
    base = wid * BPW

    pltpu.sync_copy(users_hbm.at[wid], uidx_v)
    pltpu.sync_copy(movies_hbm.at[wid], midx_v)
    pltpu.sync_copy(wb_hbm, w_v)
    pltpu.sync_copy(bb_hbm, b_v)

    mcopies = []
    for p in range(NP):
        mcopies.append(pltpu.async_copy(
            mt_hbm.at[midx_v.at[p]],
            mrows_v.at[pl.ds(p * IDX_CHUNK, IDX_CHUNK), :], sem))

    def build(s, carry):
        p = s // 8
        r = s % 8
        u16 = uidx_v[p, pl.ds(r * L, L)]
        for j in range(EMBED):
            eidx_v[j, p, pl.ds(r * L, L)] = u16 + (j * NUSERS)
        return carry
    lax.fori_loop(0, NP * 8, build, 0)

    def fire(t, carry):
        j = t // NP
        p = t % NP
        pltpu.async_copy(
            ut_hbm.at[eidx_v.at[j, p]],
            ucols_v.at[j, pl.ds(p * IDX_CHUNK, IDX_CHUNK)], esem)
        return carry
    lax.fori_loop(0, EMBED * NP, fire, 0)

    def drain(t, carry):
        j = t // NP
        p = t % NP
        pltpu.make_async_copy(
            ut_hbm.at[eidx_v.at[j, p]],
            ucols_v.at[j, pl.ds(p * IDX_CHUNK, IDX_CHUNK)], esem).wait()
        return carry
    lax.fori_loop(0, EMBED * NP, drain, 0)
    for cp in mcopies:
        cp.wait()

    lane = lax.iota(jnp.int32, L)
    bias = b_v[...]

    def chunk(c, carry):
        rows = lane + c * L
        acc = bias
        for j in range(EMBED):
            jv = jnp.full((L,), j, jnp.int32)
            uj = ucols_v[j, pl.ds(c * L, L)]
            mj = plsc.load_gather(mrows_v, [rows, jv])
            acc = acc + uj * mj * w_v[j]
        res = 4.0 / (1.0 + jnp.exp(-acc)) + 1.0
        plsc.store_scatter(o_v, [rows], res)
        return carry

    lax.fori_loop(0, NCH, chunk, 0)
    pltpu.sync_copy(o_v, out_hbm.at[pl.ds(base, BPW)])


def kernel(users, movies, user_table, movie_table, W, b):
    users3 = users.astype(jnp.int32).reshape(NW, NP, IDX_CHUNK)
    movies3 = movies.astype(jnp.int32).reshape(NW, NP, IDX_CHUNK)
    ut_flat = user_table.T.reshape(EMBED * NUSERS)
    wb = jnp.broadcast_to(W.reshape(EMBED, 1), (EMBED, L)).astype(jnp.float32)
    bb = jnp.broadcast_to(b.reshape(1), (L,)).astype(jnp.float32)

    mesh = plsc.VectorSubcoreMesh(core_axis_name="c", subcore_axis_name="s",
                                  num_cores=NC, num_subcores=NS)
    run = functools.partial(
        pl.kernel,
        out_type=jax.ShapeDtypeStruct((BATCH,), jnp.float32),
        mesh=mesh,
        scratch_types=[
            pltpu.VMEM((NP, IDX_CHUNK), jnp.int32),
            pltpu.VMEM((NP, IDX_CHUNK), jnp.int32),
            pltpu.VMEM((EMBED, NP, IDX_CHUNK), jnp.int32),
            pltpu.VMEM((EMBED, BPW), jnp.float32),
            pltpu.VMEM((BPW, EMBED), jnp.float32),
            pltpu.VMEM((EMBED, L), jnp.float32),
            pltpu.VMEM((L,), jnp.float32),
            pltpu.VMEM((BPW,), jnp.float32),
            pltpu.SemaphoreType.DMA,
            pltpu.SemaphoreType.DMA,
        ],
        compiler_params=pltpu.CompilerParams(needs_layout_passes=False,
                                             use_tc_tiling_on_sc=False),
    )(_gmf_body)
    return run(users3, movies3, ut_flat, movie_table, wb, bb)


# speedup vs baseline: 1.0023x; 1.0023x over previous
import functools

import jax
import jax.numpy as jnp
from jax import lax
from jax.experimental import pallas as pl
from jax.experimental.pallas import tpu as pltpu
from jax.experimental.pallas import tpu_sc as plsc

BATCH = 16384
EMBED = 32
NUSERS = 1000000
NC = 2
NS = 16
L = 16
NW = NC * NS
BPW = BATCH // NW
IDX_CHUNK = 128
NP = BPW // IDX_CHUNK
NCH = BPW // L


def _gmf_body(users_hbm, movies_hbm, ut_hbm, mt_hbm, wb_hbm, bb_hbm, out_hbm,
              uidx_v, midx_v, ucols_v, mrows_v, w_v, b_v, o_v,
              sem, esem):
    wid = lax.axis_index("s") * NC + lax.axis_index("c")
    base = wid * BPW

    pltpu.sync_copy(users_hbm.at[wid], uidx_v)
    pltpu.sync_copy(movies_hbm.at[wid], midx_v)
    pltpu.sync_copy(wb_hbm, w_v)
    pltpu.sync_copy(bb_hbm, b_v)

    mcopies = []
    for p in range(NP):
        mcopies.append(pltpu.async_copy(
            mt_hbm.at[midx_v.at[p]],
            mrows_v.at[pl.ds(p * IDX_CHUNK, IDX_CHUNK), :], sem))

    def fire(t, carry):
        j = t // NP
        p = t % NP
        pltpu.async_copy(
            ut_hbm.at[j].at[uidx_v.at[p]],
            ucols_v.at[j, pl.ds(p * IDX_CHUNK, IDX_CHUNK)], esem)
        return carry
    lax.fori_loop(0, EMBED * NP, fire, 0)

    def drain(t, carry):
        j = t // NP
        p = t % NP
        pltpu.make_async_copy(
            ut_hbm.at[j].at[uidx_v.at[p]],
            ucols_v.at[j, pl.ds(p * IDX_CHUNK, IDX_CHUNK)], esem).wait()
        return carry
    lax.fori_loop(0, EMBED * NP, drain, 0)
    for cp in mcopies:
        cp.wait()

    lane = lax.iota(jnp.int32, L)
    bias = b_v[...]

    def chunk(c, carry):
        rows = lane + c * L
        acc = bias
        for j in range(EMBED):
            jv = jnp.full((L,), j, jnp.int32)
            uj = ucols_v[j, pl.ds(c * L, L)]
            mj = plsc.load_gather(mrows_v, [rows, jv])
            acc = acc + uj * mj * w_v[j]
        res = 4.0 / (1.0 + jnp.exp(-acc)) + 1.0
        plsc.store_scatter(o_v, [rows], res)
        return carry

    lax.fori_loop(0, NCH, chunk, 0)
    pltpu.sync_copy(o_v, out_hbm.at[pl.ds(base, BPW)])


def kernel(users, movies, user_table, movie_table, W, b):
    users3 = users.astype(jnp.int32).reshape(NW, NP, IDX_CHUNK)
    movies3 = movies.astype(jnp.int32).reshape(NW, NP, IDX_CHUNK)
    ut_t = user_table.T
    wb = jnp.broadcast_to(W.reshape(EMBED, 1), (EMBED, L)).astype(jnp.float32)
    bb = jnp.broadcast_to(b.reshape(1), (L,)).astype(jnp.float32)

    mesh = plsc.VectorSubcoreMesh(core_axis_name="c", subcore_axis_name="s",
                                  num_cores=NC, num_subcores=NS)
    run = functools.partial(
        pl.kernel,
        out_type=jax.ShapeDtypeStruct((BATCH,), jnp.float32),
        mesh=mesh,
        scratch_types=[
            pltpu.VMEM((NP, IDX_CHUNK), jnp.int32),
            pltpu.VMEM((NP, IDX_CHUNK), jnp.int32),
            pltpu.VMEM((EMBED, BPW), jnp.float32),
            pltpu.VMEM((BPW, EMBED), jnp.float32),
            pltpu.VMEM((EMBED, L), jnp.float32),
            pltpu.VMEM((L,), jnp.float32),
            pltpu.VMEM((BPW,), jnp.float32),
            pltpu.SemaphoreType.DMA,
            pltpu.SemaphoreType.DMA,
        ],
        compiler_params=pltpu.CompilerParams(needs_layout_passes=False,
                                             use_tc_tiling_on_sc=False),
    )(_gmf_body)
    return run(users3, movies3, ut_t, movie_table, wb, bb)


# speedup vs baseline: 1.0023x; 1.0000x over previous
import functools

import jax
import jax.numpy as jnp
from jax import lax
from jax.experimental import pallas as pl
from jax.experimental.pallas import tpu as pltpu
from jax.experimental.pallas import tpu_sc as plsc
from jax.experimental import layout as jex_layout

BATCH = 16384
EMBED = 32
NUSERS = 1000000
NC = 2
NS = 16
L = 16
NW = NC * NS
BPW = BATCH // NW
IDX_CHUNK = 128
NP = BPW // IDX_CHUNK
NCH = BPW // L


def _gmf_body(users_hbm, movies_hbm, ut_hbm, mt_hbm, wb_hbm, bb_hbm, out_hbm,
              uidx_v, midx_v, ucols_v, mrows_v, w_v, b_v, o_v,
              sem, esem):
    wid = lax.axis_index("s") * NC + lax.axis_index("c")
    base = wid * BPW

    pltpu.sync_copy(users_hbm.at[wid], uidx_v)
    pltpu.sync_copy(movies_hbm.at[wid], midx_v)
    pltpu.sync_copy(wb_hbm, w_v)
    pltpu.sync_copy(bb_hbm, b_v)

    mcopies = []
    for p in range(NP):
        mcopies.append(pltpu.async_copy(
            mt_hbm.at[midx_v.at[p]],
            mrows_v.at[pl.ds(p * IDX_CHUNK, IDX_CHUNK), :], sem))

    def fire(t, carry):
        j = t // NP
        p = t % NP
        pltpu.async_copy(
            ut_hbm.at[j].at[uidx_v.at[p]],
            ucols_v.at[j, pl.ds(p * IDX_CHUNK, IDX_CHUNK)], esem)
        return carry
    lax.fori_loop(0, EMBED * NP, fire, 0)

    def drain(t, carry):
        j = t // NP
        p = t % NP
        pltpu.make_async_copy(
            ut_hbm.at[j].at[uidx_v.at[p]],
            ucols_v.at[j, pl.ds(p * IDX_CHUNK, IDX_CHUNK)], esem).wait()
        return carry
    lax.fori_loop(0, EMBED * NP, drain, 0)
    for cp in mcopies:
        cp.wait()

    lane = lax.iota(jnp.int32, L)
    bias = b_v[...]

    def chunk(c, carry):
        rows = lane + c * L
        acc = bias
        for j in range(EMBED):
            jv = jnp.full((L,), j, jnp.int32)
            uj = ucols_v[j, pl.ds(c * L, L)]
            mj = plsc.load_gather(mrows_v, [rows, jv])
            acc = acc + uj * mj * w_v[j]
        res = 4.0 / (1.0 + jnp.exp(-acc)) + 1.0
        plsc.store_scatter(o_v, [rows], res)
        return carry

    lax.fori_loop(0, NCH, chunk, 0)
    pltpu.sync_copy(o_v, out_hbm.at[pl.ds(base, BPW)])


def kernel(users, movies, user_table, movie_table, W, b):
    users3 = users.astype(jnp.int32).reshape(NW, NP, IDX_CHUNK)
    movies3 = movies.astype(jnp.int32).reshape(NW, NP, IDX_CHUNK)
    ut_lin = jax.device_put(
        user_table,
        jex_layout.Format(
            jex_layout.Layout(major_to_minor=(1, 0), tiling=((16,),)),
            jax.sharding.SingleDeviceSharding(jax.devices()[0])))
    ut_t = ut_lin.T
    wb = jnp.broadcast_to(W.reshape(EMBED, 1), (EMBED, L)).astype(jnp.float32)
    bb = jnp.broadcast_to(b.reshape(1), (L,)).astype(jnp.float32)

    mesh = plsc.VectorSubcoreMesh(core_axis_name="c", subcore_axis_name="s",
                                  num_cores=NC, num_subcores=NS)
    run = functools.partial(
        pl.kernel,
        out_type=jax.ShapeDtypeStruct((BATCH,), jnp.float32),
        mesh=mesh,
        scratch_types=[
            pltpu.VMEM((NP, IDX_CHUNK), jnp.int32),
            pltpu.VMEM((NP, IDX_CHUNK), jnp.int32),
            pltpu.VMEM((EMBED, BPW), jnp.float32),
            pltpu.VMEM((BPW, EMBED), jnp.float32),
            pltpu.VMEM((EMBED, L), jnp.float32),
            pltpu.VMEM((L,), jnp.float32),
            pltpu.VMEM((BPW,), jnp.float32),
            pltpu.SemaphoreType.DMA,
            pltpu.SemaphoreType.DMA,
        ],
        compiler_params=pltpu.CompilerParams(needs_layout_passes=False,
                                             use_tc_tiling_on_sc=False),
    )(_gmf_body)
    return run(users3, movies3, ut_t, movie_table, wb, bb)


# speedup vs baseline: 1.0027x; 1.0003x over previous
import functools

import jax
import jax.numpy as jnp
from jax import lax
from jax.experimental import pallas as pl
from jax.experimental.pallas import tpu as pltpu
from jax.experimental.pallas import tpu_sc as plsc
from jax.experimental import layout as jex_layout

BATCH = 16384
EMBED = 32
NUSERS = 1000000
NC = 2
NS = 16
L = 16
NW = NC * NS
BPW = BATCH // NW
IDX_CHUNK = 128
NP = BPW // IDX_CHUNK
NCH = BPW // L


def _gmf_body(users_hbm, movies_hbm, ut_hbm, mt_hbm, wb_hbm, bb_hbm, out_hbm,
              uidx_v, midx_v, ucols_v, mrows_v, w_v, b_v, o_v,
              sem, esem):
    wid = lax.axis_index("s") * NC + lax.axis_index("c")
    base = wid * BPW

    pltpu.sync_copy(users_hbm.at[wid], uidx_v)
    pltpu.sync_copy(movies_hbm.at[wid], midx_v)
    pltpu.sync_copy(wb_hbm, w_v)
    pltpu.sync_copy(bb_hbm, b_v)

    mcopies = []
    for p in range(NP):
        mcopies.append(pltpu.async_copy(
            mt_hbm.at[midx_v.at[p]],
            mrows_v.at[pl.ds(p * IDX_CHUNK, IDX_CHUNK), :], sem))

    def fire(t, carry):
        j = t // NP
        p = t % NP
        pltpu.async_copy(
            ut_hbm.at[j].at[uidx_v.at[p]],
            ucols_v.at[j, pl.ds(p * IDX_CHUNK, IDX_CHUNK)], esem)
        return carry
    lax.fori_loop(0, EMBED * NP, fire, 0)

    def drain(t, carry):
        j = t // NP
        p = t % NP
        pltpu.make_async_copy(
            ut_hbm.at[j].at[uidx_v.at[p]],
            ucols_v.at[j, pl.ds(p * IDX_CHUNK, IDX_CHUNK)], esem).wait()
        return carry
    lax.fori_loop(0, EMBED * NP, drain, 0)
    for cp in mcopies:
        cp.wait()

    lane = lax.iota(jnp.int32, L)
    bias = b_v[...]

    def chunk(c, carry):
        rows = lane + c * L
        acc = bias
        for j in range(EMBED):
            jv = jnp.full((L,), j, jnp.int32)
            uj = ucols_v[j, pl.ds(c * L, L)]
            mj = plsc.load_gather(mrows_v, [rows, jv])
            acc = acc + uj * mj * w_v[j]
        res = 4.0 / (1.0 + jnp.exp(-acc)) + 1.0
        plsc.store_scatter(o_v, [rows], res)
        return carry

    lax.fori_loop(0, NCH, chunk, 0)
    pltpu.sync_copy(o_v, out_hbm.at[pl.ds(base, BPW)])


def kernel(users, movies, user_table, movie_table, W, b):
    users3 = users.astype(jnp.int32).reshape(NW, NP, IDX_CHUNK)
    movies3 = movies.astype(jnp.int32).reshape(NW, NP, IDX_CHUNK)
    ut_lin = jax.device_put(
        user_table,
        jex_layout.Format(
            jex_layout.Layout(major_to_minor=(1, 0), tiling=((16,),)),
            jax.sharding.SingleDeviceSharding(jax.devices()[0])))
    ut_lin = lax.optimization_barrier(ut_lin)
    ut_t = ut_lin.T
    wb = jnp.broadcast_to(W.reshape(EMBED, 1), (EMBED, L)).astype(jnp.float32)
    bb = jnp.broadcast_to(b.reshape(1), (L,)).astype(jnp.float32)

    mesh = plsc.VectorSubcoreMesh(core_axis_name="c", subcore_axis_name="s",
                                  num_cores=NC, num_subcores=NS)
    run = functools.partial(
        pl.kernel,
        out_type=jax.ShapeDtypeStruct((BATCH,), jnp.float32),
        mesh=mesh,
        scratch_types=[
            pltpu.VMEM((NP, IDX_CHUNK), jnp.int32),
            pltpu.VMEM((NP, IDX_CHUNK), jnp.int32),
            pltpu.VMEM((EMBED, BPW), jnp.float32),
            pltpu.VMEM((BPW, EMBED), jnp.float32),
            pltpu.VMEM((EMBED, L), jnp.float32),
            pltpu.VMEM((L,), jnp.float32),
            pltpu.VMEM((BPW,), jnp.float32),
            pltpu.SemaphoreType.DMA,
            pltpu.SemaphoreType.DMA,
        ],
        compiler_params=pltpu.CompilerParams(needs_layout_passes=False,
                                             use_tc_tiling_on_sc=False),
    )(_gmf_body)
    return run(users3, movies3, ut_t, movie_table, wb, bb)


# speedup vs baseline: 14.6252x; 14.5859x over previous
import functools

import jax
import jax.numpy as jnp
from jax import lax
from jax.experimental import pallas as pl
from jax.experimental.pallas import tpu as pltpu
from jax.experimental.pallas import tpu_sc as plsc

BATCH = 16384
EMBED = 32
NUSERS = 1000000
NC = 2
NS = 16
L = 16
NW = NC * NS
BPW = BATCH // NW
IDX_CHUNK = 128
NP = BPW // IDX_CHUNK
NCH = BPW // L

AW = 32768
NWIN = NUSERS // AW
REM = NUSERS - NWIN * AW
REM_A = (REM // 128) * 128
REM_T = REM - REM_A


def _detile_body(ut_hbm, out_hbm, buf0, buf1, tbuf, sem0, sem1):
    j = lax.axis_index("s") * NC + lax.axis_index("c")
    base = j * NUSERS
    bufs = [buf0, buf1]
    sems = [sem0, sem1]

    pltpu.async_copy(ut_hbm.at[j, pl.ds(0, AW)], buf0, sem0).wait()
    for w in range(NWIN):
        nxt = (w + 1) % 2
        if w + 1 < NWIN:
            cp_in = pltpu.async_copy(
                ut_hbm.at[j, pl.ds((w + 1) * AW, AW)], bufs[nxt], sems[nxt])
        else:
            cp_in = None
        pltpu.async_copy(bufs[w % 2], out_hbm.at[pl.ds(base + w * AW, AW)],
                         sems[w % 2]).wait()
        if cp_in is not None:
            cp_in.wait()

    cb = NWIN * AW
    pltpu.sync_copy(ut_hbm.at[j, pl.ds(cb, REM_A)],
                    buf0.at[pl.ds(0, REM_A)])
    pltpu.sync_copy(buf0.at[pl.ds(0, REM_A)],
                    out_hbm.at[pl.ds(base + cb, REM_A)])
    pltpu.sync_copy(ut_hbm.at[j, pl.ds(cb + REM_A, REM_T)], tbuf)
    pltpu.sync_copy(tbuf, out_hbm.at[pl.ds(base + cb + REM_A, REM_T)])


def _gmf_body(users_hbm, movies_hbm, ut_hbm, mt_hbm, wb_hbm, bb_hbm, out_hbm,
              uidx_v, midx_v, eidx_v, ucols_v, mrows_v, w_v, b_v, o_v,
              sem, esem):
    wid = lax.axis_index("s") * NC + lax.axis_index("c")
    base = wid * BPW

    pltpu.sync_copy(users_hbm.at[wid], uidx_v)
    pltpu.sync_copy(movies_hbm.at[wid], midx_v)
    pltpu.sync_copy(wb_hbm, w_v)
    pltpu.sync_copy(bb_hbm, b_v)

    mcopies = []
    for p in range(NP):
        mcopies.append(pltpu.async_copy(
            mt_hbm.at[midx_v.at[p]],
            mrows_v.at[pl.ds(p * IDX_CHUNK, IDX_CHUNK), :], sem))

    def build(s, carry):
        p = s // 8
        r = s % 8
        u16 = uidx_v[p, pl.ds(r * L, L)]
        for j in range(EMBED):
            eidx_v[j, p, pl.ds(r * L, L)] = u16 + (j * NUSERS)
        return carry
    lax.fori_loop(0, NP * 8, build, 0)

    def fire(t, carry):
        j = t // NP
        p = t % NP
        pltpu.async_copy(
            ut_hbm.at[eidx_v.at[j, p]],
            ucols_v.at[j, pl.ds(p * IDX_CHUNK, IDX_CHUNK)], esem)
        return carry
    lax.fori_loop(0, EMBED * NP, fire, 0)

    def drain(t, carry):
        j = t // NP
        p = t % NP
        pltpu.make_async_copy(
            ut_hbm.at[eidx_v.at[j, p]],
            ucols_v.at[j, pl.ds(p * IDX_CHUNK, IDX_CHUNK)], esem).wait()
        return carry
    lax.fori_loop(0, EMBED * NP, drain, 0)
    for cp in mcopies:
        cp.wait()

    lane = lax.iota(jnp.int32, L)
    bias = b_v[...]

    def chunk(c, carry):
        rows = lane + c * L
        acc = bias
        for j in range(EMBED):
            jv = jnp.full((L,), j, jnp.int32)
            uj = ucols_v[j, pl.ds(c * L, L)]
            mj = plsc.load_gather(mrows_v, [rows, jv])
            acc = acc + uj * mj * w_v[j]
        res = 4.0 / (1.0 + jnp.exp(-acc)) + 1.0
        plsc.store_scatter(o_v, [rows], res)
        return carry

    lax.fori_loop(0, NCH, chunk, 0)
    pltpu.sync_copy(o_v, out_hbm.at[pl.ds(base, BPW)])


def kernel(users, movies, user_table, movie_table, W, b):
    users3 = users.astype(jnp.int32).reshape(NW, NP, IDX_CHUNK)
    movies3 = movies.astype(jnp.int32).reshape(NW, NP, IDX_CHUNK)
    wb = jnp.broadcast_to(W.reshape(EMBED, 1), (EMBED, L)).astype(jnp.float32)
    bb = jnp.broadcast_to(b.reshape(1), (L,)).astype(jnp.float32)

    mesh = plsc.VectorSubcoreMesh(core_axis_name="c", subcore_axis_name="s",
                                  num_cores=NC, num_subcores=NS)

    detile = functools.partial(
        pl.kernel,
        out_type=jax.ShapeDtypeStruct((EMBED * NUSERS,), jnp.float32),
        mesh=mesh,
        scratch_types=[
            pltpu.VMEM((AW,), jnp.float32),
            pltpu.VMEM((AW,), jnp.float32),
            pltpu.VMEM((REM_T,), jnp.float32),
            pltpu.SemaphoreType.DMA,
            pltpu.SemaphoreType.DMA,
        ],
        compiler_params=pltpu.CompilerParams(needs_layout_passes=False,
                                             use_tc_tiling_on_sc=True),
    )(_detile_body)
    ut_flat = detile(user_table.T)

    run = functools.partial(
        pl.kernel,
        out_type=jax.ShapeDtypeStruct((BATCH,), jnp.float32),
        mesh=mesh,
        scratch_types=[
            pltpu.VMEM((NP, IDX_CHUNK), jnp.int32),
            pltpu.VMEM((NP, IDX_CHUNK), jnp.int32),
            pltpu.VMEM((EMBED, NP, IDX_CHUNK), jnp.int32),
            pltpu.VMEM((EMBED, BPW), jnp.float32),
            pltpu.VMEM((BPW, EMBED), jnp.float32),
            pltpu.VMEM((EMBED, L), jnp.float32),
            pltpu.VMEM((L,), jnp.float32),
            pltpu.VMEM((BPW,), jnp.float32),
            pltpu.SemaphoreType.DMA,
            pltpu.SemaphoreType.DMA,
        ],
        compiler_params=pltpu.CompilerParams(needs_layout_passes=False,
                                             use_tc_tiling_on_sc=False),
    )(_gmf_body)
    return run(users3, movies3, ut_flat, movie_table, wb, bb)


# speedup vs baseline: 14.7555x; 1.0089x over previous
import functools

import jax
import jax.numpy as jnp
from jax import lax
from jax.experimental import pallas as pl
from jax.experimental.pallas import tpu as pltpu
from jax.experimental.pallas import tpu_sc as plsc

BATCH = 16384
EMBED = 32
NUSERS = 1000000
NC = 2
NS = 16
L = 16
NW = NC * NS
BPW = BATCH // NW
IDX_CHUNK = 128
NP = BPW // IDX_CHUNK
NCH = BPW // L

AW = 24576
NWIN = NUSERS // AW
NBUF = 4
REM = NUSERS - NWIN * AW
REM_A = (REM // 128) * 128
REM_T = REM - REM_A


def _detile_body(ut_hbm, out_hbm, buf0, buf1, buf2, buf3, tbuf,
                 rs0, rs1, rs2, rs3, ws0, ws1, ws2, ws3):
    j = lax.axis_index("s") * NC + lax.axis_index("c")
    base = j * NUSERS
    bufs = [buf0, buf1, buf2, buf3]
    rsem = [rs0, rs1, rs2, rs3]
    wsem = [ws0, ws1, ws2, ws3]

    def rd(w, k):
        return pltpu.async_copy(ut_hbm.at[j, pl.ds(w * AW, AW)],
                                bufs[k], rsem[k])

    def wr(w, k):
        return pltpu.async_copy(bufs[k], out_hbm.at[pl.ds(base + w * AW, AW)],
                                wsem[k])

    prim = [rd(w, w) for w in range(NBUF)]
    for w in range(NWIN):
        k = w % NBUF
        prim[k].wait()
        wcp = wr(w, k)
        if w + NBUF < NWIN:
            wcp.wait()
            prim[k] = rd(w + NBUF, k)
        else:
            wcp.wait()

    cb = NWIN * AW
    pltpu.sync_copy(ut_hbm.at[j, pl.ds(cb, REM_A)],
                    buf0.at[pl.ds(0, REM_A)])
    pltpu.sync_copy(buf0.at[pl.ds(0, REM_A)],
                    out_hbm.at[pl.ds(base + cb, REM_A)])
    pltpu.sync_copy(ut_hbm.at[j, pl.ds(cb + REM_A, REM_T)], tbuf)
    pltpu.sync_copy(tbuf, out_hbm.at[pl.ds(base + cb + REM_A, REM_T)])


def _gmf_body(users_hbm, movies_hbm, ut_hbm, mt_hbm, wb_hbm, bb_hbm, out_hbm,
              uidx_v, midx_v, eidx_v, ucols_v, mrows_v, w_v, b_v, o_v,
              sem, esem):
    wid = lax.axis_index("s") * NC + lax.axis_index("c")
    base = wid * BPW

    pltpu.sync_copy(users_hbm.at[wid], uidx_v)
    pltpu.sync_copy(movies_hbm.at[wid], midx_v)
    pltpu.sync_copy(wb_hbm, w_v)
    pltpu.sync_copy(bb_hbm, b_v)

    mcopies = []
    for p in range(NP):
        mcopies.append(pltpu.async_copy(
            mt_hbm.at[midx_v.at[p]],
            mrows_v.at[pl.ds(p * IDX_CHUNK, IDX_CHUNK), :], sem))

    def build(s, carry):
        p = s // 8
        r = s % 8
        u16 = uidx_v[p, pl.ds(r * L, L)]
        for j in range(EMBED):
            eidx_v[j, p, pl.ds(r * L, L)] = u16 + (j * NUSERS)
        return carry
    lax.fori_loop(0, NP * 8, build, 0)

    def fire(t, carry):
        j = t // NP
        p = t % NP
        pltpu.async_copy(
            ut_hbm.at[eidx_v.at[j, p]],
            ucols_v.at[j, pl.ds(p * IDX_CHUNK, IDX_CHUNK)], esem)
        return carry
    lax.fori_loop(0, EMBED * NP, fire, 0)

    def drain(t, carry):
        j = t // NP
        p = t % NP
        pltpu.make_async_copy(
            ut_hbm.at[eidx_v.at[j, p]],
            ucols_v.at[j, pl.ds(p * IDX_CHUNK, IDX_CHUNK)], esem).wait()
        return carry
    lax.fori_loop(0, EMBED * NP, drain, 0)
    for cp in mcopies:
        cp.wait()

    lane = lax.iota(jnp.int32, L)
    bias = b_v[...]

    def chunk(c, carry):
        rows = lane + c * L
        acc = bias
        for j in range(EMBED):
            jv = jnp.full((L,), j, jnp.int32)
            uj = ucols_v[j, pl.ds(c * L, L)]
            mj = plsc.load_gather(mrows_v, [rows, jv])
            acc = acc + uj * mj * w_v[j]
        res = 4.0 / (1.0 + jnp.exp(-acc)) + 1.0
        plsc.store_scatter(o_v, [rows], res)
        return carry

    lax.fori_loop(0, NCH, chunk, 0)
    pltpu.sync_copy(o_v, out_hbm.at[pl.ds(base, BPW)])


def kernel(users, movies, user_table, movie_table, W, b):
    users3 = users.astype(jnp.int32).reshape(NW, NP, IDX_CHUNK)
    movies3 = movies.astype(jnp.int32).reshape(NW, NP, IDX_CHUNK)
    wb = jnp.broadcast_to(W.reshape(EMBED, 1), (EMBED, L)).astype(jnp.float32)
    bb = jnp.broadcast_to(b.reshape(1), (L,)).astype(jnp.float32)

    mesh = plsc.VectorSubcoreMesh(core_axis_name="c", subcore_axis_name="s",
                                  num_cores=NC, num_subcores=NS)

    detile = functools.partial(
        pl.kernel,
        out_type=jax.ShapeDtypeStruct((EMBED * NUSERS,), jnp.float32),
        mesh=mesh,
        scratch_types=(
            [pltpu.VMEM((AW,), jnp.float32)] * NBUF
            + [pltpu.VMEM((REM_T,), jnp.float32)]
            + [pltpu.SemaphoreType.DMA] * (2 * NBUF)
        ),
        compiler_params=pltpu.CompilerParams(needs_layout_passes=False,
                                             use_tc_tiling_on_sc=True),
    )(_detile_body)
    ut_flat = detile(user_table.T)

    run = functools.partial(
        pl.kernel,
        out_type=jax.ShapeDtypeStruct((BATCH,), jnp.float32),
        mesh=mesh,
        scratch_types=[
            pltpu.VMEM((NP, IDX_CHUNK), jnp.int32),
            pltpu.VMEM((NP, IDX_CHUNK), jnp.int32),
            pltpu.VMEM((EMBED, NP, IDX_CHUNK), jnp.int32),
            pltpu.VMEM((EMBED, BPW), jnp.float32),
            pltpu.VMEM((BPW, EMBED), jnp.float32),
            pltpu.VMEM((EMBED, L), jnp.float32),
            pltpu.VMEM((L,), jnp.float32),
            pltpu.VMEM((BPW,), jnp.float32),
            pltpu.SemaphoreType.DMA,
            pltpu.SemaphoreType.DMA,
        ],
        compiler_params=pltpu.CompilerParams(needs_layout_passes=False,
                                             use_tc_tiling_on_sc=False),
    )(_gmf_body)
    return run(users3, movies3, ut_flat, movie_table, wb, bb)
